# Initial kernel scaffold; baseline (speedup 1.0000x reference)
import jax
import jax.numpy as jnp
from jax.experimental import pallas as pl


def kernel(input, weight, bias):
    raise NotImplementedError("write your pallas kernel here")


# speedup vs baseline: 1.4713x; 1.4713x over previous
import jax
import jax.numpy as jnp
from jax.experimental import pallas as pl

H = 224
W = 224
CIN = 96
COUT = 192
HB = 28


def _conv_body(x0_ref, x1_ref, x2_ref, w_ref, b_ref, o_ref):
    acc = jnp.zeros((HB * W, COUT), jnp.float32)
    for kh, xr in enumerate((x0_ref, x1_ref, x2_ref)):
        x = xr[0]
        for kw in range(3):
            xs = x[:, kw:kw + W, :].reshape(HB * W, CIN)
            acc += jnp.dot(xs, w_ref[kh, kw], preferred_element_type=jnp.float32)
    o_ref[...] = (acc + b_ref[...]).reshape(1, HB, W, COUT)


def kernel(input, weight, bias):
    n = input.shape[0]
    xp = jnp.pad(jnp.transpose(input, (0, 2, 3, 1)),
                 ((0, 0), (1, 1), (1, 1), (0, 0)))
    x0 = xp[:, 0:H]
    x1 = xp[:, 1:H + 1]
    x2 = xp[:, 2:H + 2]
    w = jnp.transpose(weight, (2, 3, 1, 0))
    b = bias.reshape(1, COUT)

    x_spec = pl.BlockSpec((1, HB, W + 2, CIN), lambda i, j: (i, j, 0, 0))
    out = pl.pallas_call(
        _conv_body,
        grid=(n, H // HB),
        in_specs=[
            x_spec, x_spec, x_spec,
            pl.BlockSpec((3, 3, CIN, COUT), lambda i, j: (0, 0, 0, 0)),
            pl.BlockSpec((1, COUT), lambda i, j: (0, 0)),
        ],
        out_specs=pl.BlockSpec((1, HB, W, COUT), lambda i, j: (i, j, 0, 0)),
        out_shape=jax.ShapeDtypeStruct((n, H, W, COUT), jnp.float32),
    )(x0, x1, x2, w, b)
    return jnp.transpose(out, (0, 3, 1, 2))


# speedup vs baseline: 5.8633x; 3.9851x over previous
import jax
import jax.numpy as jnp
from jax.experimental import pallas as pl
from jax.experimental.pallas import tpu as pltpu

H = 224
W = 224
CIN = 96
COUT = 192
HB = 32


def _conv_body(x_ref, top_ref, bot_ref, w_ref, b_ref, o_ref, xs):
    zcol = jnp.zeros((CIN, HB + 2, 1), jnp.float32)
    xs[:, :, 0:1] = zcol
    xs[:, :, W + 1:W + 2] = zcol
    xs[:, 0, 1:W + 1] = top_ref[0, 0]
    xs[:, 1:HB + 1, 1:W + 1] = x_ref[0]
    xs[:, HB + 1, 1:W + 1] = bot_ref[0, 0]

    b = jnp.broadcast_to(b_ref[...], (COUT, W))
    for r in range(HB):
        acc = b
        for kh in range(3):
            row = xs[:, r + kh, :]
            for kw in range(3):
                acc = acc + jnp.dot(w_ref[kh * 3 + kw], row[:, kw:kw + W],
                                    preferred_element_type=jnp.float32)
        o_ref[0, :, r, :] = acc


def kernel(input, weight, bias):
    n = input.shape[0]
    zrow = jnp.zeros((n, CIN, 1, W), jnp.float32)
    top = jnp.concatenate([zrow, input[:, :, HB - 1:H - 1:HB, :]], axis=2)
    bot = jnp.concatenate([input[:, :, HB:H:HB, :], zrow], axis=2)
    top = jnp.transpose(top, (0, 2, 1, 3))
    bot = jnp.transpose(bot, (0, 2, 1, 3))
    w = jnp.transpose(weight, (2, 3, 0, 1)).reshape(9, COUT, CIN)
    b = bias.reshape(COUT, 1)

    halo_spec = pl.BlockSpec((1, 1, CIN, W), lambda i, j: (i, j, 0, 0))
    out = pl.pallas_call(
        _conv_body,
        grid=(n, H // HB),
        in_specs=[
            pl.BlockSpec((1, CIN, HB, W), lambda i, j: (i, 0, j, 0)),
            halo_spec, halo_spec,
            pl.BlockSpec((9, COUT, CIN), lambda i, j: (0, 0, 0)),
            pl.BlockSpec((COUT, 1), lambda i, j: (0, 0)),
        ],
        out_specs=pl.BlockSpec((1, COUT, HB, W), lambda i, j: (i, 0, j, 0)),
        out_shape=jax.ShapeDtypeStruct((n, COUT, H, W), jnp.float32),
        scratch_shapes=[pltpu.VMEM((CIN, HB + 2, W + 2), jnp.float32)],
    )(input, top, bot, w, b)
    return out


# speedup vs baseline: 6.2938x; 1.0734x over previous
import jax
import jax.numpy as jnp
from jax.experimental import pallas as pl
from jax.experimental.pallas import tpu as pltpu

H = 224
W = 224
CIN = 96
COUT = 192
HB = 32


def _conv_body(x_ref, top_ref, bot_ref, w_ref, b_ref, o_ref, xs):
    zcol = jnp.zeros((HB + 2, CIN, 1), jnp.float32)
    xs[:, :, 0:1] = zcol
    xs[:, :, W + 1:W + 2] = zcol
    xs[0, :, 1:W + 1] = top_ref[0, 0]
    for r in range(HB):
        xs[1 + r, :, 1:W + 1] = x_ref[0, :, r, :]
    xs[HB + 1, :, 1:W + 1] = bot_ref[0, 0]

    ws = [w_ref[t] for t in range(9)]
    b = jnp.broadcast_to(b_ref[...], (COUT, W))
    for r in range(HB):
        acc = b
        for kh in range(3):
            row = xs[r + kh]
            for kw in range(3):
                acc = acc + jnp.dot(ws[kh * 3 + kw], row[:, kw:kw + W],
                                    preferred_element_type=jnp.float32)
        o_ref[0, :, r, :] = acc


def kernel(input, weight, bias):
    n = input.shape[0]
    zrow = jnp.zeros((n, CIN, 1, W), jnp.float32)
    top = jnp.concatenate([zrow, input[:, :, HB - 1:H - 1:HB, :]], axis=2)
    bot = jnp.concatenate([input[:, :, HB:H:HB, :], zrow], axis=2)
    top = jnp.transpose(top, (0, 2, 1, 3))
    bot = jnp.transpose(bot, (0, 2, 1, 3))
    w = jnp.transpose(weight, (2, 3, 0, 1)).reshape(9, COUT, CIN)
    b = bias.reshape(COUT, 1)

    halo_spec = pl.BlockSpec((1, 1, CIN, W), lambda i, j: (i, j, 0, 0))
    out = pl.pallas_call(
        _conv_body,
        grid=(n, H // HB),
        in_specs=[
            pl.BlockSpec((1, CIN, HB, W), lambda i, j: (i, 0, j, 0)),
            halo_spec, halo_spec,
            pl.BlockSpec((9, COUT, CIN), lambda i, j: (0, 0, 0)),
            pl.BlockSpec((COUT, 1), lambda i, j: (0, 0)),
        ],
        out_specs=pl.BlockSpec((1, COUT, HB, W), lambda i, j: (i, 0, j, 0)),
        out_shape=jax.ShapeDtypeStruct((n, COUT, H, W), jnp.float32),
        scratch_shapes=[pltpu.VMEM((HB + 2, CIN, W + 2), jnp.float32)],
    )(input, top, bot, w, b)
    return out


# speedup vs baseline: 7.7371x; 1.2293x over previous
import jax
import jax.numpy as jnp
from jax.experimental import pallas as pl
from jax.experimental.pallas import tpu as pltpu

H = 224
W = 224
CIN = 96
COUT = 192
HB = 32


def _conv_body(x_ref, top_ref, bot_ref, w_ref, b_ref, o_ref, xs):
    zcol = jnp.zeros((HB + 2, CIN, 1), jnp.bfloat16)
    xs[:, :, 0:1] = zcol
    xs[:, :, W + 1:W + 2] = zcol
    xs[0, :, 1:W + 1] = top_ref[0, 0].astype(jnp.bfloat16)
    xs[1:HB + 1, :, 1:W + 1] = jnp.swapaxes(x_ref[0], 0, 1).astype(jnp.bfloat16)
    xs[HB + 1, :, 1:W + 1] = bot_ref[0, 0].astype(jnp.bfloat16)

    ws = [w_ref[t] for t in range(3)]
    b = jnp.broadcast_to(b_ref[...], (COUT, W))
    for r in range(HB):
        acc = b
        col = xs[r:r + 3].reshape(3 * CIN, W + 2)
        for kw in range(3):
            acc = acc + jnp.dot(ws[kw], col[:, kw:kw + W],
                                preferred_element_type=jnp.float32)
        o_ref[0, :, r, :] = acc


def kernel(input, weight, bias):
    n = input.shape[0]
    zrow = jnp.zeros((n, CIN, 1, W), jnp.float32)
    top = jnp.concatenate([zrow, input[:, :, HB - 1:H - 1:HB, :]], axis=2)
    bot = jnp.concatenate([input[:, :, HB:H:HB, :], zrow], axis=2)
    top = jnp.transpose(top, (0, 2, 1, 3))
    bot = jnp.transpose(bot, (0, 2, 1, 3))
    w = jnp.transpose(weight, (3, 0, 2, 1)).reshape(3, COUT, 3 * CIN)
    w = w.astype(jnp.bfloat16)
    b = bias.reshape(COUT, 1)

    halo_spec = pl.BlockSpec((1, 1, CIN, W), lambda i, j: (i, j, 0, 0))
    out = pl.pallas_call(
        _conv_body,
        grid=(n, H // HB),
        in_specs=[
            pl.BlockSpec((1, CIN, HB, W), lambda i, j: (i, 0, j, 0)),
            halo_spec, halo_spec,
            pl.BlockSpec((3, COUT, 3 * CIN), lambda i, j: (0, 0, 0)),
            pl.BlockSpec((COUT, 1), lambda i, j: (0, 0)),
        ],
        out_specs=pl.BlockSpec((1, COUT, HB, W), lambda i, j: (i, 0, j, 0)),
        out_shape=jax.ShapeDtypeStruct((n, COUT, H, W), jnp.float32),
        scratch_shapes=[pltpu.VMEM((HB + 2, CIN, W + 2), jnp.bfloat16)],
    )(input, top, bot, w, b)
    return out


# speedup vs baseline: 8.1865x; 1.0581x over previous
import jax
import jax.numpy as jnp
from jax.experimental import pallas as pl
from jax.experimental.pallas import tpu as pltpu

H = 224
W = 224
CIN = 96
COUT = 192
HB = 56


def _conv_body(x_ref, top_ref, bot_ref, w_ref, b_ref, o_ref, xs):
    zcol = jnp.zeros((HB + 2, CIN, 1), jnp.bfloat16)
    xs[:, :, 0:1] = zcol
    xs[:, :, W + 1:W + 2] = zcol
    xs[0, :, 1:W + 1] = top_ref[0, 0].astype(jnp.bfloat16)
    xs[1:HB + 1, :, 1:W + 1] = jnp.swapaxes(x_ref[0], 0, 1).astype(jnp.bfloat16)
    xs[HB + 1, :, 1:W + 1] = bot_ref[0, 0].astype(jnp.bfloat16)

    ws = [w_ref[t] for t in range(3)]
    b = jnp.broadcast_to(b_ref[...], (COUT, W))
    for r in range(HB):
        acc = b
        col = xs[r:r + 3].reshape(3 * CIN, W + 2)
        for kw in range(3):
            acc = acc + jnp.dot(ws[kw], col[:, kw:kw + W],
                                preferred_element_type=jnp.float32)
        o_ref[0, :, r, :] = acc


def kernel(input, weight, bias):
    n = input.shape[0]
    zrow = jnp.zeros((n, CIN, 1, W), jnp.float32)
    top = jnp.concatenate([zrow, input[:, :, HB - 1:H - 1:HB, :]], axis=2)
    bot = jnp.concatenate([input[:, :, HB:H:HB, :], zrow], axis=2)
    top = jnp.transpose(top, (0, 2, 1, 3))
    bot = jnp.transpose(bot, (0, 2, 1, 3))
    w = jnp.transpose(weight, (3, 0, 2, 1)).reshape(3, COUT, 3 * CIN)
    w = w.astype(jnp.bfloat16)
    b = bias.reshape(COUT, 1)

    halo_spec = pl.BlockSpec((1, 1, CIN, W), lambda i, j: (i, j, 0, 0))
    out = pl.pallas_call(
        _conv_body,
        grid=(n, H // HB),
        in_specs=[
            pl.BlockSpec((1, CIN, HB, W), lambda i, j: (i, 0, j, 0)),
            halo_spec, halo_spec,
            pl.BlockSpec((3, COUT, 3 * CIN), lambda i, j: (0, 0, 0)),
            pl.BlockSpec((COUT, 1), lambda i, j: (0, 0)),
        ],
        out_specs=pl.BlockSpec((1, COUT, HB, W), lambda i, j: (i, 0, j, 0)),
        out_shape=jax.ShapeDtypeStruct((n, COUT, H, W), jnp.float32),
        scratch_shapes=[pltpu.VMEM((HB + 2, CIN, W + 2), jnp.bfloat16)],
    )(input, top, bot, w, b)
    return out


# speedup vs baseline: 10.2058x; 1.2467x over previous
import jax
import jax.numpy as jnp
from jax.experimental import pallas as pl
from jax.experimental.pallas import tpu as pltpu

H = 224
W = 224
CIN = 96
COUT = 192
HB = 56


def _conv_body(x_ref, top_ref, bot_ref, w_ref, b_ref, o_ref, xs):
    zc = jnp.zeros((HB + 2, CIN, 1), jnp.bfloat16)
    xs[:, 0, :, 0:1] = zc
    xs[:, 2, :, W - 1:W] = zc

    v = jnp.swapaxes(x_ref[0], 0, 1).astype(jnp.bfloat16)
    t = top_ref[0, 0].astype(jnp.bfloat16)
    u = bot_ref[0, 0].astype(jnp.bfloat16)
    xs[0, 0, :, 1:W] = t[:, 0:W - 1]
    xs[1:HB + 1, 0, :, 1:W] = v[:, :, 0:W - 1]
    xs[HB + 1, 0, :, 1:W] = u[:, 0:W - 1]
    xs[0, 1] = t
    xs[1:HB + 1, 1] = v
    xs[HB + 1, 1] = u
    xs[0, 2, :, 0:W - 1] = t[:, 1:W]
    xs[1:HB + 1, 2, :, 0:W - 1] = v[:, :, 1:W]
    xs[HB + 1, 2, :, 0:W - 1] = u[:, 1:W]

    wv = w_ref[...]
    b = jnp.broadcast_to(b_ref[...], (COUT, W))
    for r in range(HB):
        col = xs[r:r + 3].reshape(9 * CIN, W)
        o_ref[0, :, r, :] = b + jnp.dot(wv, col,
                                        preferred_element_type=jnp.float32)


def kernel(input, weight, bias):
    n = input.shape[0]
    zrow = jnp.zeros((n, CIN, 1, W), jnp.float32)
    top = jnp.concatenate([zrow, input[:, :, HB - 1:H - 1:HB, :]], axis=2)
    bot = jnp.concatenate([input[:, :, HB:H:HB, :], zrow], axis=2)
    top = jnp.transpose(top, (0, 2, 1, 3))
    bot = jnp.transpose(bot, (0, 2, 1, 3))
    w = jnp.transpose(weight, (0, 2, 3, 1)).reshape(COUT, 9 * CIN)
    w = w.astype(jnp.bfloat16)
    b = bias.reshape(COUT, 1)

    halo_spec = pl.BlockSpec((1, 1, CIN, W), lambda i, j: (i, j, 0, 0))
    out = pl.pallas_call(
        _conv_body,
        grid=(n, H // HB),
        in_specs=[
            pl.BlockSpec((1, CIN, HB, W), lambda i, j: (i, 0, j, 0)),
            halo_spec, halo_spec,
            pl.BlockSpec((COUT, 9 * CIN), lambda i, j: (0, 0)),
            pl.BlockSpec((COUT, 1), lambda i, j: (0, 0)),
        ],
        out_specs=pl.BlockSpec((1, COUT, HB, W), lambda i, j: (i, 0, j, 0)),
        out_shape=jax.ShapeDtypeStruct((n, COUT, H, W), jnp.float32),
        scratch_shapes=[pltpu.VMEM((HB + 2, 3, CIN, W), jnp.bfloat16)],
    )(input, top, bot, w, b)
    return out
